# baseline (speedup 1.0000x reference)
import functools

import jax
import jax.numpy as jnp
from jax import lax
from jax.experimental import pallas as pl
from jax.experimental.pallas import tpu as pltpu
from jax.experimental.pallas import tpu_sc as plsc

BATCH = 4096
SEQ = 50
DIM = 64
VOCAB = 1000000
NC, NS = 2, 16
NW = NC * NS
G = BATCH // NW
L = 16
NJG = G // L

R = 256
NBLK = 1954
OFF = 1953 * R
TAIL_BLK = (VOCAB - OFF) // R
PACKED_ROWS = NBLK * R


def _repack_body(lo_ref, hi_ref, out_ref):
    eye = jnp.eye(DIM, dtype=jnp.float32)
    dn = (((0,), (0,)), ((), ()))
    out_ref[:, 0:DIM] = jax.lax.dot_general(
        lo_ref[...], eye, dn, precision=jax.lax.Precision.HIGHEST
    )
    out_ref[:, DIM : 2 * DIM] = jax.lax.dot_general(
        hi_ref[...], eye, dn, precision=jax.lax.Precision.HIGHEST
    )


def _repack(tbl_t):
    def lo_map(b):
        return (0, jnp.where(b == NBLK - 1, 2 * (NBLK - 1), b))

    def hi_map(b):
        return (0, jnp.where(b == NBLK - 1, 0, (NBLK - 1) + b))

    return pl.pallas_call(
        _repack_body,
        grid=(NBLK,),
        in_specs=[
            pl.BlockSpec((DIM, R), lo_map),
            pl.BlockSpec((DIM, R), hi_map),
        ],
        out_specs=pl.BlockSpec((R, 2 * DIM), lambda b: (b, 0)),
        out_shape=jax.ShapeDtypeStruct((PACKED_ROWS, 2 * DIM), jnp.float32),
    )(tbl_t, tbl_t)


def _start_gather(tbl_hbm, vrow_v, rows_v, gsem, s, buf):
    pltpu.async_copy(tbl_hbm.at[vrow_v.at[s]], rows_v.at[buf], gsem.at[buf])


def _wait_gather(tbl_hbm, vrow_v, rows_v, gsem, buf):
    pltpu.make_async_copy(
        tbl_hbm.at[vrow_v.at[0]], rows_v.at[buf], gsem.at[buf]
    ).wait()


def _body(idx_hbm, tbl_hbm, out_hbm, idx_v, vrow_v, rows_v, obuf, colbuf, gsem):
    wid = lax.axis_index("s") * NC + lax.axis_index("c")
    base = wid * G
    pltpu.sync_copy(idx_hbm.at[:, pl.ds(base, G)], idx_v)

    def prep(s, _):
        for jg in range(NJG):
            v = idx_v[s, pl.ds(jg * L, L)]
            ge1 = (v >= OFF).astype(jnp.int32)
            vrow_v[s, pl.ds(jg * L, L)] = v - ge1 * OFF
        return ()

    lax.fori_loop(0, SEQ, prep, ())

    jrows = [jnp.arange(L, dtype=jnp.int32) + jg * L for jg in range(NJG)]
    dvecs = [
        jnp.bitwise_and(jnp.arange(L, dtype=jnp.int32) + k, L - 1)
        for k in range(L)
    ]

    def process(s, buf):
        rowbuf = rows_v.at[buf]
        for jg in range(NJG):
            v = idx_v[s, pl.ds(jg * L, L)]
            sel = jnp.logical_and(v >= OFF, v < 2 * OFF).astype(jnp.int32)
            colbuf[pl.ds(jg * L, L)] = lax.shift_left(sel, 6)

        def dloop(t, _):
            d0 = t * L
            cbd = [colbuf[pl.ds(jg * L, L)] + d0 for jg in range(NJG)]
            for k in range(L):
                dvd = dvecs[k] + d0
                for jg in range(NJG):
                    col = cbd[jg] + dvecs[k]
                    val = plsc.load_gather(rowbuf, [jrows[jg], col])
                    plsc.store_scatter(obuf, [dvd, jrows[jg]], val)
            return ()

        lax.fori_loop(0, DIM // L, dloop, ())
        pltpu.sync_copy(obuf, out_hbm.at[s, :, pl.ds(base, G)])

    _start_gather(tbl_hbm, vrow_v, rows_v, gsem, 0, 0)

    def group(g, _):
        s0 = 2 * g
        _wait_gather(tbl_hbm, vrow_v, rows_v, gsem, 0)
        _start_gather(tbl_hbm, vrow_v, rows_v, gsem, s0 + 1, 1)
        process(s0, 0)

        s1 = s0 + 1
        _wait_gather(tbl_hbm, vrow_v, rows_v, gsem, 1)

        @pl.when(s1 + 1 < SEQ)
        def _():
            _start_gather(tbl_hbm, vrow_v, rows_v, gsem, s1 + 1, 0)

        process(s1, 1)
        return ()

    lax.fori_loop(0, SEQ // 2, group, ())


@jax.jit
def kernel(indices, table):
    idx_t = indices.T.astype(jnp.int32)
    tbl_t = table.T
    tbl2 = _repack(tbl_t)
    mesh = plsc.VectorSubcoreMesh(core_axis_name="c", subcore_axis_name="s")
    run = pl.kernel(
        _body,
        out_type=jax.ShapeDtypeStruct((SEQ, DIM, BATCH), jnp.float32),
        mesh=mesh,
        scratch_types=[
            pltpu.VMEM((SEQ, G), jnp.int32),
            pltpu.VMEM((SEQ, G), jnp.int32),
            pltpu.VMEM((2, G, 2 * DIM), jnp.float32),
            pltpu.VMEM((DIM, G), jnp.float32),
            pltpu.VMEM((G,), jnp.int32),
            pltpu.SemaphoreType.DMA((2,)),
        ],
        compiler_params=pltpu.CompilerParams(
            use_tc_tiling_on_sc=True, needs_layout_passes=False
        ),
    )
    out = run(idx_t, tbl2)
    return out.transpose(2, 0, 1)


# speedup vs baseline: 2.2757x; 2.2757x over previous
import functools

import jax
import jax.numpy as jnp
from jax import lax
from jax.experimental import pallas as pl
from jax.experimental.pallas import tpu as pltpu
from jax.experimental.pallas import tpu_sc as plsc

BATCH = 4096
SEQ = 50
DIM = 64
VOCAB = 1000000
NC, NS = 2, 16
NW = NC * NS
G = BATCH // NW
L = 16


def _start_gather(tbl_hbm, idx_v, rows_v, gsem, s, buf):
    pltpu.async_copy(tbl_hbm.at[idx_v.at[s]], rows_v.at[buf], gsem.at[buf])


def _wait_gather(tbl_hbm, idx_v, rows_v, gsem, buf):
    pltpu.make_async_copy(
        tbl_hbm.at[idx_v.at[0]], rows_v.at[buf], gsem.at[buf]
    ).wait()


def _body(idx_hbm, tbl_hbm, out_hbm, idx_v, rows_v, obuf, gsem):
    wid = lax.axis_index("s") * NC + lax.axis_index("c")
    base = wid * G
    pltpu.sync_copy(idx_hbm.at[:, pl.ds(base, G)], idx_v)

    jrows = [jnp.arange(L, dtype=jnp.int32) + jg * L for jg in range(L // 2)]
    dvecs = [
        jnp.bitwise_and(jnp.arange(L, dtype=jnp.int32) + k, L - 1)
        for k in range(L)
    ]

    def process(s, buf):
        rowbuf = rows_v.at[buf]

        def dloop(t, _):
            d0 = t * L
            for k in range(L):
                dvd = dvecs[k] + d0
                for jg in range(G // L):
                    val = plsc.load_gather(rowbuf, [jrows[jg], dvd])
                    plsc.store_scatter(obuf, [dvd, jrows[jg]], val)
            return ()

        lax.fori_loop(0, DIM // L, dloop, ())
        pltpu.sync_copy(obuf, out_hbm.at[s, :, pl.ds(base, G)])

    _start_gather(tbl_hbm, idx_v, rows_v, gsem, 0, 0)

    def group(g, _):
        s0 = 2 * g
        _wait_gather(tbl_hbm, idx_v, rows_v, gsem, 0)
        _start_gather(tbl_hbm, idx_v, rows_v, gsem, s0 + 1, 1)
        process(s0, 0)

        s1 = s0 + 1
        _wait_gather(tbl_hbm, idx_v, rows_v, gsem, 1)

        @pl.when(s1 + 1 < SEQ)
        def _():
            _start_gather(tbl_hbm, idx_v, rows_v, gsem, s1 + 1, 0)

        process(s1, 1)
        return ()

    lax.fori_loop(0, SEQ // 2, group, ())


@jax.jit
def kernel(indices, table):
    idx_t = indices.T.astype(jnp.int32)
    tbl2 = jnp.pad(table[:VOCAB], ((0, 0), (0, DIM)))
    mesh = plsc.VectorSubcoreMesh(core_axis_name="c", subcore_axis_name="s")
    run = pl.kernel(
        _body,
        out_type=jax.ShapeDtypeStruct((SEQ, DIM, BATCH), jnp.float32),
        mesh=mesh,
        scratch_types=[
            pltpu.VMEM((SEQ, G), jnp.int32),
            pltpu.VMEM((2, G, 2 * DIM), jnp.float32),
            pltpu.VMEM((DIM, G), jnp.float32),
            pltpu.SemaphoreType.DMA((2,)),
        ],
        compiler_params=pltpu.CompilerParams(
            use_tc_tiling_on_sc=True, needs_layout_passes=False
        ),
    )
    out = run(idx_t, tbl2)
    return out.transpose(2, 0, 1)


# speedup vs baseline: 2.3344x; 1.0258x over previous
import functools

import jax
import jax.numpy as jnp
from jax import lax
from jax.experimental import pallas as pl
from jax.experimental.pallas import tpu as pltpu
from jax.experimental.pallas import tpu_sc as plsc

BATCH = 4096
SEQ = 50
DIM = 64
VOCAB = 1000000
NC, NS = 2, 16
NW = NC * NS
G = BATCH // NW
L = 16


def _start_gather(tbl_hbm, idx_v, rows_v, gsem, s, buf):
    pltpu.async_copy(tbl_hbm.at[idx_v.at[s]], rows_v.at[buf], gsem.at[buf])


def _wait_gather(tbl_hbm, idx_v, rows_v, gsem, buf):
    pltpu.make_async_copy(
        tbl_hbm.at[idx_v.at[0]], rows_v.at[buf], gsem.at[buf]
    ).wait()


def _body(idx_hbm, tbl_hbm, out_hbm, idx_v, rows_v, obuf, gsem, osem):
    wid = lax.axis_index("s") * NC + lax.axis_index("c")
    base = wid * G
    pltpu.sync_copy(idx_hbm.at[:, pl.ds(base, G)], idx_v)

    jrows = [jnp.arange(L, dtype=jnp.int32) + jg * L for jg in range(L // 2)]
    dvecs = [
        jnp.bitwise_and(jnp.arange(L, dtype=jnp.int32) + k, L - 1)
        for k in range(L)
    ]

    def process(s, buf):
        rowbuf = rows_v.at[buf]
        ob = obuf.at[buf]

        @pl.when(s >= 2)
        def _():
            pltpu.make_async_copy(
                ob, out_hbm.at[0, :, pl.ds(base, G)], osem.at[buf]
            ).wait()

        def dloop(t, _):
            d0 = t * L
            for k in range(L):
                dvd = dvecs[k] + d0
                for jg in range(G // L):
                    val = plsc.load_gather(rowbuf, [jrows[jg], dvd])
                    plsc.store_scatter(ob, [dvd, jrows[jg]], val)
            return ()

        lax.fori_loop(0, DIM // L, dloop, ())
        pltpu.async_copy(ob, out_hbm.at[s, :, pl.ds(base, G)], osem.at[buf])

    _start_gather(tbl_hbm, idx_v, rows_v, gsem, 0, 0)

    def group(g, _):
        s0 = 2 * g
        _wait_gather(tbl_hbm, idx_v, rows_v, gsem, 0)
        _start_gather(tbl_hbm, idx_v, rows_v, gsem, s0 + 1, 1)
        process(s0, 0)

        s1 = s0 + 1
        _wait_gather(tbl_hbm, idx_v, rows_v, gsem, 1)

        @pl.when(s1 + 1 < SEQ)
        def _():
            _start_gather(tbl_hbm, idx_v, rows_v, gsem, s1 + 1, 0)

        process(s1, 1)
        return ()

    lax.fori_loop(0, SEQ // 2, group, ())

    for b in range(2):
        pltpu.make_async_copy(
            obuf.at[b], out_hbm.at[0, :, pl.ds(base, G)], osem.at[b]
        ).wait()


@jax.jit
def kernel(indices, table):
    idx_t = indices.T.astype(jnp.int32)
    tbl2 = jnp.pad(table[:VOCAB], ((0, 0), (0, DIM)))
    mesh = plsc.VectorSubcoreMesh(core_axis_name="c", subcore_axis_name="s")
    run = pl.kernel(
        _body,
        out_type=jax.ShapeDtypeStruct((SEQ, DIM, BATCH), jnp.float32),
        mesh=mesh,
        scratch_types=[
            pltpu.VMEM((SEQ, G), jnp.int32),
            pltpu.VMEM((2, G, 2 * DIM), jnp.float32),
            pltpu.VMEM((2, DIM, G), jnp.float32),
            pltpu.SemaphoreType.DMA((2,)),
            pltpu.SemaphoreType.DMA((2,)),
        ],
        compiler_params=pltpu.CompilerParams(
            use_tc_tiling_on_sc=True, needs_layout_passes=False
        ),
    )
    out = run(idx_t, tbl2)
    return out.transpose(2, 0, 1)
